# baseline (speedup 1.0000x reference)
import functools

import jax
import jax.numpy as jnp
from jax import lax
from jax.experimental import pallas as pl
from jax.experimental.pallas import tpu as pltpu, tpu_sc as plsc

INPUT_DIM = 26
NUM_EMB = 100000
OUT_DIM = 32
LANES = 16
PAD_W = 128

ROWS_PER_CHUNK = 16
LOOKUPS = ROWS_PER_CHUNK * INPUT_DIM


def _build(num_rows):
    NC, NS = 2, 16
    NW = NC * NS
    rows_per_w = num_rows // NW
    chunks = rows_per_w // ROWS_PER_CHUNK
    assert chunks % 2 == 0

    mesh = plsc.VectorSubcoreMesh(core_axis_name="c", subcore_axis_name="s")

    @functools.partial(
        pl.kernel,
        mesh=mesh,
        out_type=jax.ShapeDtypeStruct((num_rows * OUT_DIM,), jnp.float32),
        scratch_types=[
            pltpu.VMEM((LOOKUPS,), jnp.int32),
            pltpu.VMEM((LOOKUPS,), jnp.int32),
            pltpu.VMEM((LOOKUPS,), jnp.int32),
            pltpu.VMEM((LOOKUPS, PAD_W), jnp.float32),
            pltpu.VMEM((LOOKUPS, PAD_W), jnp.float32),
            pltpu.VMEM((ROWS_PER_CHUNK * OUT_DIM,), jnp.float32),
            pltpu.SemaphoreType.DMA,
            pltpu.SemaphoreType.DMA,
        ],
    )
    def k(tab_hbm, idx_hbm, offs_hbm, out_hbm,
          idx_a, idx_b, offs_v, rows_a, rows_b, out_v, sem_a, sem_b):
        wid = lax.axis_index("s") * NC + lax.axis_index("c")
        pltpu.sync_copy(offs_hbm, offs_v)
        idx_base = wid * (rows_per_w * INPUT_DIM)
        out_base = wid * (rows_per_w * OUT_DIM)

        def stage(g, idx_v, rows_v, sem):
            off = pl.multiple_of(idx_base + g * LOOKUPS, 8)
            pltpu.sync_copy(idx_hbm.at[pl.ds(off, LOOKUPS)], idx_v)

            def offs_body(j, _):
                sl = pl.ds(j * LANES, LANES)
                idx_v[sl] = idx_v[sl] + offs_v[sl]
                return 0

            lax.fori_loop(0, LOOKUPS // LANES, offs_body, 0)
            return pltpu.async_copy(tab_hbm.at[idx_v], rows_v, sem)

        def process(g, rows_v):
            def acc_body(c, _):
                base = c * INPUT_DIM
                a0 = rows_v[base, pl.ds(0, LANES)]
                a1 = rows_v[base, pl.ds(LANES, LANES)]
                for i in range(1, INPUT_DIM):
                    a0 = a0 + rows_v[base + i, pl.ds(0, LANES)]
                    a1 = a1 + rows_v[base + i, pl.ds(LANES, LANES)]
                out_v[pl.ds(c * OUT_DIM, LANES)] = a0
                out_v[pl.ds(c * OUT_DIM + LANES, LANES)] = a1
                return 0

            lax.fori_loop(0, ROWS_PER_CHUNK, acc_body, 0)
            off = pl.multiple_of(out_base + g * (ROWS_PER_CHUNK * OUT_DIM), 8)
            pltpu.sync_copy(
                out_v, out_hbm.at[pl.ds(off, ROWS_PER_CHUNK * OUT_DIM)]
            )

        stage(0, idx_a, rows_a, sem_a)

        def pair_body(t, _):
            g = 2 * t
            cp_b = stage(g + 1, idx_b, rows_b, sem_b)
            pltpu.make_async_copy(tab_hbm.at[idx_a], rows_a, sem_a).wait()
            process(g, rows_a)

            @pl.when(g + 2 < chunks)
            def _():
                stage(g + 2, idx_a, rows_a, sem_a)

            cp_b.wait()
            process(g + 1, rows_b)
            return 0

        lax.fori_loop(0, chunks // 2, pair_body, 0)

    return k


def kernel(int_vec, tables):
    bs, num_nodes, input_dim = int_vec.shape
    num_rows = bs * num_nodes
    tab2d = tables.reshape(input_dim * tables.shape[1], tables.shape[2])
    tab_wide = jnp.pad(tab2d, ((0, 0), (0, PAD_W - OUT_DIM)))
    idx_flat = int_vec.reshape(num_rows * input_dim)
    offs = jnp.tile(
        jnp.arange(INPUT_DIM, dtype=jnp.int32) * NUM_EMB, ROWS_PER_CHUNK
    )
    out = _build(num_rows)(tab_wide, idx_flat, offs)
    return out.reshape(bs, num_nodes, tables.shape[2])
